# Initial kernel scaffold; baseline (speedup 1.0000x reference)
import jax
import jax.numpy as jnp
from jax.experimental import pallas as pl


def kernel(features, memory_bank, influence_weight, distance_weight):
    raise NotImplementedError("write your pallas kernel here")


# speedup vs baseline: 4.8903x; 4.8903x over previous
import functools

import jax
import jax.numpy as jnp
from jax import lax
from jax.experimental import pallas as pl
from jax.experimental.pallas import tpu as pltpu
from jax.experimental.pallas import tpu_sc as plsc

K = 9
MIN_STD = 0.01
MAX_STD = 0.5

_RB = 256
_CB = 2048


def _topk_body(a_ref, b_ref, outd_ref, outi_ref, runs_ref, runi_ref):
    t = pl.program_id(1)
    nt = pl.num_programs(1)
    a = a_ref[...]
    b = b_ref[...]

    @pl.when(t == 0)
    def _init():
        runs_ref[...] = jnp.full((_RB, 128), jnp.inf, jnp.float32)
        runi_ref[...] = jnp.zeros((_RB, 128), jnp.int32)

    ab = lax.dot_general(a, b, (((1,), (1,)), ((), ())),
                         preferred_element_type=jnp.float32)
    a2 = jnp.sum(a * a, axis=1, keepdims=True)
    b2 = jnp.sum(b * b, axis=1)[None, :]
    d2 = a2 + b2 - 2.0 * ab
    d = jnp.sqrt(jnp.maximum(d2, 0.0))

    run_s = runs_ref[...]
    run_i = runi_ref[...]
    cand = jnp.concatenate([d, run_s], axis=1)
    lane = lax.broadcasted_iota(jnp.int32, (_RB, _CB + 128), 1)
    tile_idx = lax.broadcasted_iota(jnp.int32, (_RB, _CB), 1) + t * _CB
    cidx = jnp.concatenate([tile_idx, run_i], axis=1)

    vals, idxs = [], []
    for _ in range(K):
        mv = jnp.min(cand, axis=1, keepdims=True)
        eq = cand == mv
        pos = jnp.min(jnp.where(eq, lane, jnp.int32(1 << 30)), axis=1,
                      keepdims=True)
        sel = lane == pos
        gi = jnp.max(jnp.where(sel, cidx, -1), axis=1, keepdims=True)
        vals.append(mv)
        idxs.append(gi)
        cand = jnp.where(sel, jnp.inf, cand)

    new_s = jnp.concatenate(
        vals + [jnp.full((_RB, 128 - K), jnp.inf, jnp.float32)], axis=1)
    new_i = jnp.concatenate(
        idxs + [jnp.zeros((_RB, 128 - K), jnp.int32)], axis=1)
    runs_ref[...] = new_s
    runi_ref[...] = new_i

    @pl.when(t == nt - 1)
    def _out():
        lane128 = lax.broadcasted_iota(jnp.int32, (_RB, 128), 1)
        valid = lane128 < K
        dmean = jnp.sum(jnp.where(valid, new_s, 0.0), axis=1,
                        keepdims=True) * (1.0 / K)
        outd_ref[...] = jnp.where(valid, new_s, dmean)
        outi_ref[...] = new_i


def _run_topk(flat, bank):
    n = flat.shape[0]
    v = bank.shape[0]
    grid = (n // _RB, v // _CB)
    return pl.pallas_call(
        _topk_body,
        grid=grid,
        in_specs=[
            pl.BlockSpec((_RB, flat.shape[1]), lambda r, t: (r, 0)),
            pl.BlockSpec((_CB, flat.shape[1]), lambda r, t: (t, 0)),
        ],
        out_specs=[
            pl.BlockSpec((_RB, 128), lambda r, t: (r, 0)),
            pl.BlockSpec((_RB, 128), lambda r, t: (r, 0)),
        ],
        out_shape=[
            jax.ShapeDtypeStruct((n, 128), jnp.float32),
            jax.ShapeDtypeStruct((n, 128), jnp.int32),
        ],
        scratch_shapes=[
            pltpu.VMEM((_RB, 128), jnp.float32),
            pltpu.VMEM((_RB, 128), jnp.int32),
        ],
        compiler_params=pltpu.CompilerParams(
            dimension_semantics=("arbitrary", "arbitrary")),
    )(flat, bank)


_NW = 32
_RW = 64
_G = 8
_NG = _RW // _G


def _sc_influence_body(bank_hbm, feat_hbm, idx_hbm, d16_hbm, iw_hbm, out_hbm,
                       idx_v, w16_v, iw_v, gath, f_v, o_v, sem):
    c = lax.axis_index("c")
    s = lax.axis_index("s")
    wid = s * 2 + c
    rowbase = wid * _RW

    pltpu.sync_copy(idx_hbm.at[pl.ds(wid * _RW * K, _RW * K)], idx_v)
    pltpu.sync_copy(d16_hbm.at[pl.ds(rowbase, _RW)], w16_v)
    pltpu.sync_copy(iw_hbm, iw_v)
    lane = lax.broadcasted_iota(jnp.int32, (16,), 0)

    def _wbody(i, carry):
        dv = w16_v[i, :]
        w16_v[i, :] = jnp.where(lane < K, 1.0 / (float(K) * dv), 0.0)
        return carry

    lax.fori_loop(0, _RW, _wbody, 0)

    for g in range(_NG):
        pltpu.async_copy(bank_hbm.at[idx_v.at[pl.ds(g * _G * K, _G * K)]],
                         gath, sem).wait()
        pltpu.sync_copy(feat_hbm.at[pl.ds(rowbase + g * _G, _G)], f_v)
        for r in range(_G):
            wrow = w16_v[g * _G + r, :]
            wb = [jnp.full((16,), wrow[j], jnp.float32) for j in range(K)]
            ws = wb[0]
            for j in range(1, K):
                ws = ws + wb[j]

            def _cbody(ci, carry, r=r, wb=wb, ws=ws):
                sl = pl.ds(ci * 16, 16)
                acc = f_v[r, sl] * ws
                for j in range(K):
                    acc = acc - gath[r * K + j, sl] * wb[j]
                o_v[r, sl] = jnp.abs(acc) * iw_v[sl]
                return carry

            lax.fori_loop(0, 1024 // 16, _cbody, 0)
        pltpu.sync_copy(o_v, out_hbm.at[pl.ds(rowbase + g * _G, _G)])


def _run_sc_influence(bank, flat, idx_flat, d16, iw):
    n, dim = flat.shape
    mesh = plsc.VectorSubcoreMesh(core_axis_name="c", subcore_axis_name="s")
    fn = functools.partial(
        pl.kernel,
        mesh=mesh,
        out_type=jax.ShapeDtypeStruct((n, dim), jnp.float32),
        scratch_types=[
            pltpu.VMEM((_RW * K,), jnp.int32),
            pltpu.VMEM((_RW, 16), jnp.float32),
            pltpu.VMEM((dim,), jnp.float32),
            pltpu.VMEM((_G * K, dim), jnp.float32),
            pltpu.VMEM((_G, dim), jnp.float32),
            pltpu.VMEM((_G, dim), jnp.float32),
            pltpu.SemaphoreType.DMA,
        ],
    )(_sc_influence_body)
    return fn(bank, flat, idx_flat, d16, iw)


def _noise_body(x_ref, dn_ref, out_ref):
    x = x_ref[...]
    dim = x.shape[1]
    m = jnp.mean(x, axis=1, keepdims=True)
    xc = x - m
    var = jnp.sum(xc * xc, axis=1, keepdims=True) * (1.0 / (dim - 1))
    inorm = xc / (jnp.sqrt(var) + 1e-8)
    comb = inorm + dn_ref[:, :1]
    out_ref[...] = MIN_STD + (MAX_STD - MIN_STD) * jax.nn.sigmoid(comb)


def _run_noise(infl, dn_b):
    n, dim = infl.shape
    return pl.pallas_call(
        _noise_body,
        grid=(n // _RB,),
        in_specs=[
            pl.BlockSpec((_RB, dim), lambda r: (r, 0)),
            pl.BlockSpec((_RB, 128), lambda r: (r, 0)),
        ],
        out_specs=pl.BlockSpec((_RB, dim), lambda r: (r, 0)),
        out_shape=jax.ShapeDtypeStruct((n, dim), jnp.float32),
    )(infl, dn_b)


def kernel(features, memory_bank, influence_weight, distance_weight):
    B, N, D = features.shape
    n = B * N
    flat = features.reshape(n, D)

    outd, outi = _run_topk(flat, memory_bank)
    d9 = outd[:, :K]
    rm = outd[:, K]

    gm = jnp.mean(rm)
    gvar = jnp.sum((rm - gm) ** 2) * D / (n * D - 1)
    dn = (rm - gm) / (jnp.sqrt(gvar) + 1e-8)
    dn_b = jnp.broadcast_to((distance_weight[0] * dn)[:, None], (n, 128))

    idx_flat = outi[:, :K].reshape(-1)
    infl = _run_sc_influence(memory_bank, flat, idx_flat, outd[:, :16],
                             influence_weight)
    noise = _run_noise(infl, dn_b)

    return (infl.reshape(B, N, D), noise.reshape(B, N, D),
            d9.reshape(B, N, K))


# speedup vs baseline: 5.9997x; 1.2269x over previous
import functools

import jax
import jax.numpy as jnp
from jax import lax
from jax.experimental import pallas as pl
from jax.experimental.pallas import tpu as pltpu
from jax.experimental.pallas import tpu_sc as plsc

K = 9
MIN_STD = 0.01
MAX_STD = 0.5

_RB = 256
_CB = 2048


_DEPTH = 4
_TK = 3


def _bank_sq_body(b_ref, out_ref):
    b = b_ref[...]
    out_ref[...] = jnp.sum(b * b, axis=1)[None, None, :]


def _run_bank_sq(bank):
    v = bank.shape[0]
    nt = v // _CB
    return pl.pallas_call(
        _bank_sq_body,
        grid=(nt,),
        in_specs=[pl.BlockSpec((_CB, bank.shape[1]), lambda t: (t, 0))],
        out_specs=pl.BlockSpec((1, 1, _CB), lambda t: (t, 0, 0)),
        out_shape=jax.ShapeDtypeStruct((nt, 1, _CB), jnp.float32),
    )(bank)


def _topk_body(a_ref, b_ref, b2_ref, outd_ref, outi_ref, *scr):
    rv_ref = scr[:_DEPTH]
    iv_ref = scr[_DEPTH:]
    t = pl.program_id(1)
    nt = pl.num_programs(1)
    ncol = _CB // 128

    @pl.when(t == 0)
    def _init():
        for i in range(_DEPTH):
            rv_ref[i][...] = jnp.full((_RB, 128), jnp.inf, jnp.float32)
            iv_ref[i][...] = jnp.zeros((_RB, 128), jnp.int32)

    ab = lax.dot_general(a_ref[...], b_ref[...], (((1,), (1,)), ((), ())),
                         preferred_element_type=jnp.float32)
    s = b2_ref[...].reshape(1, _CB) - 2.0 * ab

    sv = s.reshape(_RB, ncol, 128)
    idx3d = (lax.broadcasted_iota(jnp.int32, (_RB, ncol, 128), 1) * 128
             + lax.broadcasted_iota(jnp.int32, (_RB, ncol, 128), 2)
             + t * _CB)
    tv, ti = [], []
    for i in range(_TK):
        m = jnp.min(sv, axis=1)
        eq = sv == m[:, None, :]
        gi = jnp.min(jnp.where(eq, idx3d, jnp.int32(1 << 30)), axis=1)
        if i < _TK - 1:
            sv = jnp.where(eq, jnp.inf, sv)
        tv.append(m)
        ti.append(gi)

    rv = [r[...] for r in rv_ref]
    iv = [r[...] for r in iv_ref]
    for m, gi in zip(tv, ti):
        new, nidx = m, gi
        for k in range(_DEPTH):
            cmp = new < rv[k]
            if k < _DEPTH - 1:
                nv = jnp.where(cmp, rv[k], new)
                ni = jnp.where(cmp, iv[k], nidx)
            rv[k] = jnp.where(cmp, new, rv[k])
            iv[k] = jnp.where(cmp, nidx, iv[k])
            if k < _DEPTH - 1:
                new, nidx = nv, ni
    for i in range(_DEPTH):
        rv_ref[i][...] = rv[i]
        iv_ref[i][...] = iv[i]

    @pl.when(t == nt - 1)
    def _out():
        w = _DEPTH * 128
        cand = jnp.concatenate([r[...] for r in rv_ref], axis=1)
        cidx_c = jnp.concatenate([r[...] for r in iv_ref], axis=1)
        lanew = lax.broadcasted_iota(jnp.int32, (_RB, w), 1)
        lane9 = lax.broadcasted_iota(jnp.int32, (_RB, 128), 1)

        def _extract(k, carry):
            cc, dacc, iacc = carry
            mv = jnp.min(cc, axis=1, keepdims=True)
            eq = cc == mv
            pos = jnp.min(jnp.where(eq, lanew, jnp.int32(1 << 30)), axis=1,
                          keepdims=True)
            sel = lanew == pos
            gi = jnp.max(jnp.where(sel, cidx_c, -1), axis=1, keepdims=True)
            cc = jnp.where(sel, jnp.inf, cc)
            tgt = lane9 == k
            dacc = jnp.where(tgt, mv, dacc)
            iacc = jnp.where(tgt, gi, iacc)
            return cc, dacc, iacc

        _, dacc, iacc = lax.fori_loop(
            0, K, _extract,
            (cand, jnp.zeros((_RB, 128), jnp.float32),
             jnp.zeros((_RB, 128), jnp.int32)))
        a2 = jnp.sum(a_ref[...] * a_ref[...], axis=1, keepdims=True)
        valid = lane9 < K
        d9 = jnp.sqrt(jnp.maximum(dacc + a2, 0.0))
        dmean = jnp.sum(jnp.where(valid, d9, 0.0), axis=1,
                        keepdims=True) * (1.0 / K)
        outd_ref[...] = jnp.where(valid, d9, dmean)
        outi_ref[...] = iacc


def _run_topk(flat, bank, b2):
    n = flat.shape[0]
    v = bank.shape[0]
    grid = (n // _RB, v // _CB)
    return pl.pallas_call(
        _topk_body,
        grid=grid,
        in_specs=[
            pl.BlockSpec((_RB, flat.shape[1]), lambda r, t: (r, 0)),
            pl.BlockSpec((_CB, flat.shape[1]), lambda r, t: (t, 0)),
            pl.BlockSpec((1, 1, _CB), lambda r, t: (t, 0, 0)),
        ],
        out_specs=[
            pl.BlockSpec((_RB, 128), lambda r, t: (r, 0)),
            pl.BlockSpec((_RB, 128), lambda r, t: (r, 0)),
        ],
        out_shape=[
            jax.ShapeDtypeStruct((n, 128), jnp.float32),
            jax.ShapeDtypeStruct((n, 128), jnp.int32),
        ],
        scratch_shapes=(
            [pltpu.VMEM((_RB, 128), jnp.float32) for _ in range(_DEPTH)]
            + [pltpu.VMEM((_RB, 128), jnp.int32) for _ in range(_DEPTH)]),
        compiler_params=pltpu.CompilerParams(
            dimension_semantics=("arbitrary", "arbitrary")),
    )(flat, bank, b2)


_NW = 32
_RW = 64
_G = 8
_NG = _RW // _G


def _sc_influence_body(bank_hbm, feat_hbm, idx_hbm, d16_hbm, iw_hbm, out_hbm,
                       idx_v, w16_v, iw_v, gath, f_v, o_v, sem):
    c = lax.axis_index("c")
    s = lax.axis_index("s")
    wid = s * 2 + c
    rowbase = wid * _RW

    pltpu.sync_copy(idx_hbm.at[pl.ds(wid * _RW * K, _RW * K)], idx_v)
    pltpu.sync_copy(d16_hbm.at[pl.ds(rowbase, _RW)], w16_v)
    pltpu.sync_copy(iw_hbm, iw_v)
    lane = lax.broadcasted_iota(jnp.int32, (16,), 0)

    def _wbody(i, carry):
        dv = w16_v[i, :]
        w16_v[i, :] = jnp.where(lane < K, 1.0 / (float(K) * dv), 0.0)
        return carry

    lax.fori_loop(0, _RW, _wbody, 0)

    for g in range(_NG):
        pltpu.async_copy(bank_hbm.at[idx_v.at[pl.ds(g * _G * K, _G * K)]],
                         gath, sem).wait()
        pltpu.sync_copy(feat_hbm.at[pl.ds(rowbase + g * _G, _G)], f_v)
        for r in range(_G):
            wrow = w16_v[g * _G + r, :]
            wb = [jnp.full((16,), wrow[j], jnp.float32) for j in range(K)]
            ws = wb[0]
            for j in range(1, K):
                ws = ws + wb[j]

            def _cbody(ci, carry, r=r, wb=wb, ws=ws):
                sl = pl.ds(ci * 16, 16)
                acc = f_v[r, sl] * ws
                for j in range(K):
                    acc = acc - gath[r * K + j, sl] * wb[j]
                o_v[r, sl] = jnp.abs(acc) * iw_v[sl]
                return carry

            lax.fori_loop(0, 1024 // 16, _cbody, 0)
        pltpu.sync_copy(o_v, out_hbm.at[pl.ds(rowbase + g * _G, _G)])


def _run_sc_influence(bank, flat, idx_flat, d16, iw):
    n, dim = flat.shape
    mesh = plsc.VectorSubcoreMesh(core_axis_name="c", subcore_axis_name="s")
    fn = functools.partial(
        pl.kernel,
        mesh=mesh,
        out_type=jax.ShapeDtypeStruct((n, dim), jnp.float32),
        scratch_types=[
            pltpu.VMEM((_RW * K,), jnp.int32),
            pltpu.VMEM((_RW, 16), jnp.float32),
            pltpu.VMEM((dim,), jnp.float32),
            pltpu.VMEM((_G * K, dim), jnp.float32),
            pltpu.VMEM((_G, dim), jnp.float32),
            pltpu.VMEM((_G, dim), jnp.float32),
            pltpu.SemaphoreType.DMA,
        ],
    )(_sc_influence_body)
    return fn(bank, flat, idx_flat, d16, iw)


def _noise_body(x_ref, dn_ref, out_ref):
    x = x_ref[...]
    dim = x.shape[1]
    m = jnp.mean(x, axis=1, keepdims=True)
    xc = x - m
    var = jnp.sum(xc * xc, axis=1, keepdims=True) * (1.0 / (dim - 1))
    inorm = xc / (jnp.sqrt(var) + 1e-8)
    comb = inorm + dn_ref[:, :1]
    out_ref[...] = MIN_STD + (MAX_STD - MIN_STD) * jax.nn.sigmoid(comb)


def _run_noise(infl, dn_b):
    n, dim = infl.shape
    return pl.pallas_call(
        _noise_body,
        grid=(n // _RB,),
        in_specs=[
            pl.BlockSpec((_RB, dim), lambda r: (r, 0)),
            pl.BlockSpec((_RB, 128), lambda r: (r, 0)),
        ],
        out_specs=pl.BlockSpec((_RB, dim), lambda r: (r, 0)),
        out_shape=jax.ShapeDtypeStruct((n, dim), jnp.float32),
    )(infl, dn_b)


def kernel(features, memory_bank, influence_weight, distance_weight):
    B, N, D = features.shape
    n = B * N
    flat = features.reshape(n, D)

    b2 = _run_bank_sq(memory_bank)
    outd, outi = _run_topk(flat, memory_bank, b2)
    d9 = outd[:, :K]
    rm = outd[:, K]

    gm = jnp.mean(rm)
    gvar = jnp.sum((rm - gm) ** 2) * D / (n * D - 1)
    dn = (rm - gm) / (jnp.sqrt(gvar) + 1e-8)
    dn_b = jnp.broadcast_to((distance_weight[0] * dn)[:, None], (n, 128))

    idx_flat = outi[:, :K].reshape(-1)
    infl = _run_sc_influence(memory_bank, flat, idx_flat, outd[:, :16],
                             influence_weight)
    noise = _run_noise(infl, dn_b)

    return (infl.reshape(B, N, D), noise.reshape(B, N, D),
            d9.reshape(B, N, K))
